# baseline (speedup 1.0000x reference)
import jax
import jax.numpy as jnp
from jax.experimental import pallas as pl
from jax.experimental.pallas import tpu as pltpu

_NROWS = 21504
_STAGE = 128
_BIG = 4096


def _bcast_kernel(logps_ref, out_ref, bufa, sem_a, sem_h):
    rows = out_ref.reshape(_NROWS, 1000)
    bufa[...] = jnp.broadcast_to(logps_ref[...], bufa.shape)
    cp = pltpu.make_async_copy(bufa, rows.at[pl.ds(0, _STAGE), :], sem_a)
    cp.start()
    cp.wait()
    staged = _STAGE
    while staged < _BIG:
        cp = pltpu.make_async_copy(
            rows.at[pl.ds(0, staged), :],
            rows.at[pl.ds(staged, staged), :],
            sem_h,
        )
        cp.start()
        cp.wait()
        staged *= 2
    offs = []
    off = staged
    while off < _NROWS:
        n = min(_BIG, _NROWS - off)
        offs.append((off, n))
        off += n
    for off, n in offs:
        pltpu.make_async_copy(
            rows.at[pl.ds(0, n), :], rows.at[pl.ds(off, n), :], sem_h
        ).start()
    for off, n in offs:
        pltpu.make_async_copy(
            rows.at[pl.ds(0, n), :], rows.at[pl.ds(off, n), :], sem_h
        ).wait()


def kernel(hist, logps):
    S, B = hist.shape
    V = logps.shape[0]
    logps2d = logps.reshape(1, V)

    out = pl.pallas_call(
        _bcast_kernel,
        in_specs=[pl.BlockSpec((1, V), lambda: (0, 0))],
        out_specs=pl.BlockSpec(memory_space=pltpu.MemorySpace.HBM),
        out_shape=jax.ShapeDtypeStruct((S + 1, B, V), jnp.float32),
        scratch_shapes=[
            pltpu.VMEM((_STAGE, 1000), jnp.float32),
            pltpu.SemaphoreType.DMA,
            pltpu.SemaphoreType.DMA,
        ],
    )(logps2d)
    return out


# speedup vs baseline: 18.1707x; 18.1707x over previous
import functools

import jax
import jax.numpy as jnp
from jax import lax
from jax.experimental import pallas as pl
from jax.experimental.pallas import tpu as pltpu
from jax.experimental.pallas import tpu_sc as plsc

_CH = 128
_HALF = 512


def kernel(hist, logps):
    S, B = hist.shape
    V = logps.shape[0]
    nslab = S + 1
    logps2d = logps.reshape(1, V)

    mesh = plsc.ScalarSubcoreMesh(axis_name="c")

    @functools.partial(
        pl.kernel,
        out_type=jax.ShapeDtypeStruct((nslab, B, V), jnp.float32),
        mesh=mesh,
        scratch_types=[
            pltpu.VMEM_SHARED((_CH, V), jnp.float32),
            pltpu.SemaphoreType.DMA,
            pltpu.SemaphoreType.DMA,
        ],
    )
    def _bcast(logps_hbm, out_hbm, shared, sem_fill, sem_out):
        c = lax.axis_index("c")
        base = c * _HALF
        for r in range(_CH):
            pltpu.make_async_copy(
                logps_hbm, shared.at[pl.ds(r, 1)], sem_fill
            ).start()
        for r in range(_CH):
            pltpu.make_async_copy(
                logps_hbm, shared.at[pl.ds(r, 1)], sem_fill
            ).wait()
        for i in range(nslab):
            for j in range(_HALF // _CH):
                pltpu.make_async_copy(
                    shared,
                    out_hbm.at[i, pl.ds(base + j * _CH, _CH), :],
                    sem_out,
                ).start()
        for i in range(nslab):
            for j in range(_HALF // _CH):
                pltpu.make_async_copy(
                    shared,
                    out_hbm.at[i, pl.ds(base + j * _CH, _CH), :],
                    sem_out,
                ).wait()

    return _bcast(logps2d)


# speedup vs baseline: 25.8179x; 1.4209x over previous
import jax
import jax.numpy as jnp
from jax.experimental import pallas as pl
from jax.experimental.pallas import tpu as pltpu

_NCOPY = 21
_NPRI = 2


def _bcast_kernel(logps_ref, out_ref, bufa, sems):
    out_rows = out_ref.reshape(21504, 1000)
    bufa[...] = jnp.broadcast_to(logps_ref[...], bufa.shape)
    cps = []
    for i in range(_NCOPY):
        cps.append(
            pltpu.async_copy(
                bufa,
                out_rows.at[pl.ds(i * 1024, 1024), :],
                sems.at[i % _NPRI],
                priority=i % _NPRI,
            )
        )
    for cp in cps:
        cp.wait()


def kernel(hist, logps):
    S, B = hist.shape
    V = logps.shape[0]
    logps2d = logps.reshape(1, V)

    out = pl.pallas_call(
        _bcast_kernel,
        in_specs=[pl.BlockSpec((1, V), lambda: (0, 0))],
        out_specs=pl.BlockSpec(memory_space=pltpu.MemorySpace.HBM),
        out_shape=jax.ShapeDtypeStruct((S + 1, B, V), jnp.float32),
        scratch_shapes=[
            pltpu.VMEM((1024, 1000), jnp.float32),
            pltpu.SemaphoreType.DMA((_NPRI,)),
        ],
    )(logps2d)
    return out


# speedup vs baseline: 25.9110x; 1.0036x over previous
import jax
import jax.numpy as jnp
from jax.experimental import pallas as pl


def _broadcast_kernel(logps_ref, out_ref):
    out_ref[...] = jnp.broadcast_to(logps_ref[...][:, None, :], out_ref.shape)


def kernel(hist, logps):
    S, B = hist.shape
    V = logps.shape[0]
    logps2d = logps.reshape(1, V)

    out = pl.pallas_call(
        _broadcast_kernel,
        grid=(S + 1,),
        in_specs=[pl.BlockSpec((1, V), lambda i: (0, 0))],
        out_specs=pl.BlockSpec((1, B, V), lambda i: (i, 0, 0)),
        out_shape=jax.ShapeDtypeStruct((S + 1, B, V), jnp.float32),
    )(logps2d)
    return out
